# baseline (speedup 1.0000x reference)
import functools

import jax
import jax.numpy as jnp
from jax import lax
from jax.experimental import pallas as pl
from jax.experimental.pallas import tpu as pltpu
from jax.experimental.pallas import tpu_sc as plsc

VOCAB = 1000000
EMBED = 64
BATCH = 16384
NEG = 5
NIDX = NEG + 1
NROW = NIDX + 1

NC, NS, LANES = 2, 16, 16
NW = NC * NS
BPW = BATCH // NW
CB = 64
NCHUNK = BPW // CB
NGROUP = CB // LANES

TBLK = VOCAB // 128
BPT = TBLK // NW
TAIL = VOCAB - TBLK * 128
_SC_PARAMS = pltpu.CompilerParams(needs_layout_passes=False)


def _sc_transpose_w(w_t, w_tail):
    mesh = plsc.VectorSubcoreMesh(core_axis_name="c", subcore_axis_name="s")

    @functools.partial(
        pl.kernel,
        out_type=jax.ShapeDtypeStruct((VOCAB, EMBED), jnp.float32),
        mesh=mesh,
        compiler_params=_SC_PARAMS,
        scratch_types=[
            pltpu.VMEM((2, EMBED, 128), jnp.float32),
            pltpu.VMEM((2, EMBED, 128), jnp.float32),
            pltpu.VMEM((2, 128, EMBED), jnp.float32),
            pltpu.VMEM((2, 128, EMBED), jnp.float32),
            pltpu.VMEM((EMBED, EMBED), jnp.float32),
            pltpu.SemaphoreType.DMA,
            pltpu.SemaphoreType.DMA,
            pltpu.SemaphoreType.DMA,
            pltpu.SemaphoreType.DMA,
        ],
    )
    def k(wt_hbm, tail_hbm, out_hbm, in0, in1, out0, out1, tail_v,
          si0, si1, so0, so1):
        wid = lax.axis_index("s") * NC + lax.axis_index("c")
        lane = lax.iota(jnp.int32, LANES)
        ins, outs = (in0, in1), (out0, out1)
        sis, sos = (si0, si1), (so0, so1)

        def blk_off(b):
            return pl.multiple_of((wid + NW * b) * 128, 128)

        def enq_in(g, par):
            for q in range(2):
                b = g * 2 + q

                @pl.when(b < BPT)
                def _():
                    pltpu.async_copy(wt_hbm.at[:, pl.ds(blk_off(b), 128)],
                                     ins[par].at[q], sis[par])

        def drain_in(par):
            for _ in range(2):
                pltpu.make_async_copy(wt_hbm.at[:, pl.ds(0, 128)],
                                      ins[par].at[0], sis[par]).wait()

        def transpose(par):
            for q in range(2):
                q_vec = jnp.full((LANES,), q, jnp.int32)

                def drow(d, _, q=q, q_vec=q_vec):
                    d_vec = jnp.full((LANES,), d, jnp.int32)
                    for gr in range(8):
                        val = ins[par][q, d, pl.ds(gr * LANES, LANES)]
                        plsc.store_scatter(
                            outs[par], [q_vec, gr * LANES + lane, d_vec], val)
                    return 0

                lax.fori_loop(0, EMBED, drow, 0)

        def enq_out(g, par):
            for q in range(2):
                b = g * 2 + q
                pltpu.async_copy(outs[par].at[q],
                                 out_hbm.at[pl.ds(blk_off(b), 128)],
                                 sos[par])

        def drain_out(par):
            for _ in range(2):
                pltpu.make_async_copy(outs[par].at[0],
                                      out_hbm.at[pl.ds(0, 128)],
                                      sos[par]).wait()

        enq_in(0, 0)
        enq_in(1, 1)

        def round_(gg, _):
            for par in range(2):
                g = gg * 2 + par
                drain_in(par)

                @pl.when(gg > 0)
                def _():
                    drain_out(par)

                transpose(par)
                enq_out(g, par)
                enq_in(g + 2, par)
            return 0

        lax.fori_loop(0, BPT // 4, round_, 0)
        for par in range(2):
            drain_out(par)

        @pl.when(wid < TBLK - BPT * NW)
        def _():
            off = pl.multiple_of((BPT * NW + wid) * 128, 128)
            pltpu.async_copy(wt_hbm.at[:, pl.ds(off, 128)], in0.at[0],
                             si0).wait()

            def drow(d, _):
                d_vec = jnp.full((LANES,), d, jnp.int32)
                for gr in range(8):
                    val = in0[0, d, pl.ds(gr * LANES, LANES)]
                    plsc.store_scatter(
                        out0, [jnp.zeros((LANES,), jnp.int32),
                               gr * LANES + lane, d_vec], val)
                return 0

            lax.fori_loop(0, EMBED, drow, 0)
            pltpu.sync_copy(out0.at[0], out_hbm.at[pl.ds(off, 128)])

        @pl.when(wid == 7)
        def _():
            pltpu.sync_copy(tail_hbm, tail_v)
            pltpu.sync_copy(tail_v, out_hbm.at[pl.ds(TBLK * 128, TAIL)])

    return k(w_t, w_tail)


def _sc_scores(pos_w, vidx, w_table, v_table):
    mesh = plsc.VectorSubcoreMesh(core_axis_name="c", subcore_axis_name="s")

    @functools.partial(
        pl.kernel,
        out_type=jax.ShapeDtypeStruct((NW, NCHUNK, NIDX, CB), jnp.float32),
        mesh=mesh,
        compiler_params=_SC_PARAMS,
        scratch_types=[
            pltpu.VMEM((BPW,), jnp.int32),
            pltpu.VMEM((NIDX, BPW), jnp.int32),
            pltpu.VMEM((2, CB, EMBED), jnp.float32),
            pltpu.VMEM((2, NIDX, CB, EMBED), jnp.float32),
            pltpu.VMEM((NIDX, CB), jnp.float32),
            pltpu.SemaphoreType.DMA,
            pltpu.SemaphoreType.DMA,
        ],
    )
    def k(pos_w_hbm, vidx_hbm, w_hbm, v_hbm, out_hbm,
          widx_v, vidx_v, wrows, vrows, scores_v, sem0, sem1):
        wid = lax.axis_index("s") * NC + lax.axis_index("c")
        lane = lax.iota(jnp.int32, LANES)
        zero = jnp.zeros((LANES,), jnp.float32)
        sems = (sem0, sem1)
        base = wid * BPW

        icps = [pltpu.async_copy(pos_w_hbm.at[pl.ds(base, BPW)], widx_v,
                                 sem0)]
        for j in range(NIDX):
            icps.append(pltpu.async_copy(vidx_hbm.at[j, pl.ds(base, BPW)],
                                         vidx_v.at[j], sem0))
        for cp in icps:
            cp.wait()

        def enqueue(c, buf):
            sem = sems[buf]

            def enq(g, _):
                off = c * CB + g * LANES
                wvec = widx_v[pl.ds(off, LANES)]
                vvecs = [vidx_v[j, pl.ds(off, LANES)] for j in range(NIDX)]
                for l in range(LANES):
                    i = g * LANES + l
                    pltpu.async_copy(w_hbm.at[wvec[l]], wrows.at[buf, i],
                                     sem)
                    for j in range(NIDX):
                        pltpu.async_copy(v_hbm.at[vvecs[j][l]],
                                         vrows.at[buf, j, i], sem)
                return 0

            lax.fori_loop(0, NGROUP, enq, 0)

        def drain(buf):
            def one(i, _):
                pltpu.make_async_copy(w_hbm.at[0], wrows.at[buf, 0],
                                      sems[buf]).wait()
                return 0

            lax.fori_loop(0, NROW * CB, one, 0)

        def compute(c, buf):
            for g in range(NGROUP):
                sl = pl.ds(g * LANES, LANES)
                i_vec = jnp.full((LANES,), g * LANES, jnp.int32) + lane
                b_vec = jnp.full((LANES,), buf, jnp.int32)

                def body(d, accs, i_vec=i_vec, b_vec=b_vec):
                    d_vec = jnp.full((LANES,), d, jnp.int32)
                    wv = plsc.load_gather(wrows, [b_vec, i_vec, d_vec])
                    return tuple(
                        accs[j] + wv * plsc.load_gather(
                            vrows,
                            [b_vec, jnp.full((LANES,), j, jnp.int32), i_vec,
                             d_vec])
                        for j in range(NIDX))

                accs = lax.fori_loop(0, EMBED, body, (zero,) * NIDX)
                scores_v[0, sl] = accs[0]
                for j in range(1, NIDX):
                    scores_v[j, sl] = -accs[j]

            pltpu.sync_copy(scores_v, out_hbm.at[wid, c])

        enqueue(0, 0)
        for c in range(NCHUNK):
            if c + 1 < NCHUNK:
                enqueue(c + 1, (c + 1) % 2)
            drain(c % 2)
            compute(c, c % 2)

    return k(pos_w, vidx, w_table, v_table)


def _tc_loss_body(x_ref, o_ref):
    x = jnp.clip(x_ref[...], -10.0, 10.0)
    o_ref[0, 0] = -jnp.sum(jax.nn.log_sigmoid(x))


def kernel(pos_w, pos_v, neg_v, w_embeddings, v_embeddings):
    pos_w = jnp.asarray(pos_w, jnp.int32)
    vidx = jnp.concatenate(
        [jnp.asarray(pos_v, jnp.int32)[None, :],
         jnp.asarray(neg_v, jnp.int32).T], axis=0)

    wrm = _sc_transpose_w(w_embeddings.T, w_embeddings[TBLK * 128:])

    scores = _sc_scores(pos_w, vidx, wrm, v_embeddings)
    flat = scores.reshape(BATCH * NIDX // 128, 128)

    loss = pl.pallas_call(
        _tc_loss_body,
        out_shape=jax.ShapeDtypeStruct((1, 1), jnp.float32),
        out_specs=pl.BlockSpec(memory_space=pltpu.SMEM),
    )(flat)
    return loss[0, 0]
